# baseline (speedup 1.0000x reference)
import functools

import jax
import jax.numpy as jnp
from jax import lax
from jax.experimental import pallas as pl
from jax.experimental.pallas import tpu as pltpu
from jax.experimental.pallas import tpu_sc as plsc

N = 16384
E = 64
K = 2
NS = 16
LANES = 16
ROWS_PER_TILE = N // NS
SEL_PER_TILE = (N * K) // NS
SCALE = float(E) / (float(N) * K * N)
ECH = E // LANES
NCHUNK = 4
CHUNK_ROWS = ROWS_PER_TILE // NCHUNK


_mesh = plsc.VectorSubcoreMesh(
    core_axis_name="c", subcore_axis_name="s", num_cores=1, num_subcores=NS
)


@functools.partial(
    pl.kernel,
    out_type=jax.ShapeDtypeStruct((LANES,), jnp.float32),
    mesh=_mesh,
    compiler_params=pltpu.CompilerParams(needs_layout_passes=False),
    scratch_types=[
        pltpu.VMEM((ROWS_PER_TILE * E,), jnp.float32),
        pltpu.VMEM((SEL_PER_TILE,), jnp.int32),
        pltpu.VMEM((E,), jnp.float32),
        pltpu.VMEM((NS * E,), jnp.float32),
        pltpu.VMEM((LANES,), jnp.float32),
        pltpu.VMEM((NS * LANES,), jnp.float32),
        pltpu.VMEM_SHARED((NS * E,), jnp.float32),
        pltpu.VMEM_SHARED((NS * LANES,), jnp.float32),
    ],
)
def _lb_loss_kernel(probs_hbm, sel_hbm, out_hbm,
                    buf_v, sel_v, col_v, colall_v, vec_v, accall_v,
                    shared_col, shared_acc):
    s = lax.axis_index("s")

    pltpu.sync_copy(
        probs_hbm.at[pl.ds(s * (ROWS_PER_TILE * E), ROWS_PER_TILE * E)],
        buf_v)
    pltpu.sync_copy(
        sel_hbm.at[pl.ds(s * SEL_PER_TILE, SEL_PER_TILE)], sel_v)

    @plsc.parallel_loop(
        0, ROWS_PER_TILE,
        carry=tuple(jnp.zeros((LANES,), jnp.float32) for _ in range(ECH)),
        unroll=8)
    def accs(i, a):
        return tuple(a[j] + buf_v[pl.ds(i * E + j * LANES, LANES)]
                     for j in range(ECH))

    for j in range(ECH):
        col_v[pl.ds(j * LANES, LANES)] = accs[j]

    pltpu.sync_copy(col_v, shared_col.at[pl.ds(s * E, E)])
    plsc.subcore_barrier()
    pltpu.sync_copy(shared_col, colall_v)
    for j in range(ECH):
        cj = colall_v[pl.ds(j * LANES, LANES)]
        for t in range(1, NS):
            cj = cj + colall_v[pl.ds(t * E + j * LANES, LANES)]
        col_v[pl.ds(j * LANES, LANES)] = cj

    @plsc.parallel_loop(0, SEL_PER_TILE // LANES,
                        carry=jnp.zeros((LANES,), jnp.float32), unroll=8)
    def acc(i, a):
        idx = sel_v[pl.ds(i * LANES, LANES)]
        return a + plsc.load_gather(col_v, [idx])

    vec_v[...] = acc
    pltpu.sync_copy(vec_v, shared_acc.at[pl.ds(s * LANES, LANES)])
    plsc.subcore_barrier()

    @pl.when(s == 0)
    def _finish():
        pltpu.sync_copy(shared_acc, accall_v)
        tot = accall_v[pl.ds(0, LANES)]
        for t in range(1, NS):
            tot = tot + accall_v[pl.ds(t * LANES, LANES)]
        total = jnp.sum(tot) * SCALE
        vec_v[...] = jnp.full((LANES,), total, jnp.float32)
        pltpu.sync_copy(vec_v, out_hbm)


def kernel(router_probs, selected_experts):
    sel_flat = selected_experts.astype(jnp.int32).reshape(-1)
    out = _lb_loss_kernel(router_probs.reshape(-1), sel_flat)
    return out[0] * 1.0


# speedup vs baseline: 1.2936x; 1.2936x over previous
import functools

import jax
import jax.numpy as jnp
from jax import lax
from jax.experimental import pallas as pl
from jax.experimental.pallas import tpu as pltpu
from jax.experimental.pallas import tpu_sc as plsc

N = 16384
E = 64
K = 2
NS = 16
LANES = 16
ROWS_PER_TILE = N // NS
SEL_PER_TILE = (N * K) // NS
SCALE = float(E) / (float(N) * K * N)
ECH = E // LANES
NCHUNK = 4
CHUNK_ROWS = ROWS_PER_TILE // NCHUNK


_mesh = plsc.VectorSubcoreMesh(
    core_axis_name="c", subcore_axis_name="s", num_cores=1, num_subcores=NS
)


@functools.partial(
    pl.kernel,
    out_type=jax.ShapeDtypeStruct((LANES,), jnp.float32),
    mesh=_mesh,
    compiler_params=pltpu.CompilerParams(needs_layout_passes=False),
    scratch_types=[
        pltpu.VMEM((ROWS_PER_TILE * E,), jnp.float32),
        pltpu.VMEM((SEL_PER_TILE,), jnp.int32),
        pltpu.VMEM((E,), jnp.float32),
        pltpu.VMEM((NS * E,), jnp.float32),
        pltpu.VMEM((LANES,), jnp.float32),
        pltpu.VMEM((NS * LANES,), jnp.float32),
        pltpu.VMEM_SHARED((NS * E,), jnp.float32),
        pltpu.VMEM_SHARED((NS * LANES,), jnp.float32),
    ],
)
def _lb_loss_kernel(probs_hbm, sel0_hbm, sel1_hbm, out_hbm,
                    buf_v, sel_v, col_v, colall_v, vec_v, accall_v,
                    shared_col, shared_acc):
    s = lax.axis_index("s")
    half = SEL_PER_TILE // 2

    pltpu.sync_copy(
        probs_hbm.at[pl.ds(s * (ROWS_PER_TILE * E), ROWS_PER_TILE * E)],
        buf_v)
    pltpu.sync_copy(sel0_hbm.at[pl.ds(s * half, half)],
                    sel_v.at[pl.ds(0, half)])
    pltpu.sync_copy(sel1_hbm.at[pl.ds(s * half, half)],
                    sel_v.at[pl.ds(half, half)])

    @plsc.parallel_loop(
        0, ROWS_PER_TILE,
        carry=tuple(jnp.zeros((LANES,), jnp.float32) for _ in range(ECH)),
        unroll=8)
    def accs(i, a):
        return tuple(a[j] + buf_v[pl.ds(i * E + j * LANES, LANES)]
                     for j in range(ECH))

    for j in range(ECH):
        col_v[pl.ds(j * LANES, LANES)] = accs[j]

    pltpu.sync_copy(col_v, shared_col.at[pl.ds(s * E, E)])
    plsc.subcore_barrier()
    pltpu.sync_copy(shared_col, colall_v)
    for j in range(ECH):
        cj = colall_v[pl.ds(j * LANES, LANES)]
        for t in range(1, NS):
            cj = cj + colall_v[pl.ds(t * E + j * LANES, LANES)]
        col_v[pl.ds(j * LANES, LANES)] = cj

    @plsc.parallel_loop(0, SEL_PER_TILE // LANES,
                        carry=jnp.zeros((LANES,), jnp.float32), unroll=8)
    def acc(i, a):
        idx = sel_v[pl.ds(i * LANES, LANES)]
        return a + plsc.load_gather(col_v, [idx])

    vec_v[...] = acc
    pltpu.sync_copy(vec_v, shared_acc.at[pl.ds(s * LANES, LANES)])
    plsc.subcore_barrier()

    @pl.when(s == 0)
    def _finish():
        pltpu.sync_copy(shared_acc, accall_v)
        tot = accall_v[pl.ds(0, LANES)]
        for t in range(1, NS):
            tot = tot + accall_v[pl.ds(t * LANES, LANES)]
        total = jnp.sum(tot) * SCALE
        vec_v[...] = jnp.full((LANES,), total, jnp.float32)
        pltpu.sync_copy(vec_v, out_hbm)


def kernel(router_probs, selected_experts):
    sel = selected_experts.astype(jnp.int32)
    out = _lb_loss_kernel(router_probs.reshape(-1), sel[:, 0], sel[:, 1])
    return out[0] * 1.0


# speedup vs baseline: 1.3074x; 1.0106x over previous
import functools

import jax
import jax.numpy as jnp
from jax import lax
from jax.experimental import pallas as pl
from jax.experimental.pallas import tpu as pltpu
from jax.experimental.pallas import tpu_sc as plsc

N = 16384
E = 64
K = 2
NS = 16
LANES = 16
SEL_PER_TILE = (N * K) // NS
SCALE = float(E) / (float(N) * K * N)
ECH = E // LANES
BR = 1024
SUB = 8


def _colsum_tc_body(probs_ref, out_ref):
    i = pl.program_id(0)

    @pl.when(i == 0)
    def _init():
        out_ref[...] = jnp.zeros_like(out_ref)

    blk = probs_ref[...].reshape(BR // SUB, SUB, E)
    out_ref[...] += jnp.sum(blk, axis=0)


_colsum_tc = pl.pallas_call(
    _colsum_tc_body,
    grid=(N // BR,),
    in_specs=[pl.BlockSpec((BR, E), lambda i: (i, 0))],
    out_specs=pl.BlockSpec((SUB, E), lambda i: (0, 0)),
    out_shape=jax.ShapeDtypeStruct((SUB, E), jnp.float32),
)


_mesh = plsc.VectorSubcoreMesh(
    core_axis_name="c", subcore_axis_name="s", num_cores=1, num_subcores=NS
)


@functools.partial(
    pl.kernel,
    out_type=jax.ShapeDtypeStruct((LANES,), jnp.float32),
    mesh=_mesh,
    compiler_params=pltpu.CompilerParams(needs_layout_passes=False),
    scratch_types=[
        pltpu.VMEM((SUB * E,), jnp.float32),
        pltpu.VMEM((SEL_PER_TILE,), jnp.int32),
        pltpu.VMEM((E,), jnp.float32),
        pltpu.VMEM((LANES,), jnp.float32),
        pltpu.VMEM((NS * LANES,), jnp.float32),
        pltpu.VMEM_SHARED((NS * LANES,), jnp.float32),
    ],
)
def _lb_gather_kernel(colsum8_hbm, sel0_hbm, sel1_hbm, out_hbm,
                      col8_v, sel_v, col_v, vec_v, accall_v, shared_acc):
    s = lax.axis_index("s")
    half = SEL_PER_TILE // 2

    pltpu.sync_copy(colsum8_hbm, col8_v)
    pltpu.sync_copy(sel0_hbm.at[pl.ds(s * half, half)],
                    sel_v.at[pl.ds(0, half)])
    pltpu.sync_copy(sel1_hbm.at[pl.ds(s * half, half)],
                    sel_v.at[pl.ds(half, half)])

    for j in range(ECH):
        cj = col8_v[pl.ds(j * LANES, LANES)]
        for r in range(1, SUB):
            cj = cj + col8_v[pl.ds(r * E + j * LANES, LANES)]
        col_v[pl.ds(j * LANES, LANES)] = cj

    @plsc.parallel_loop(0, SEL_PER_TILE // LANES,
                        carry=jnp.zeros((LANES,), jnp.float32), unroll=8)
    def acc(i, a):
        idx = sel_v[pl.ds(i * LANES, LANES)]
        return a + plsc.load_gather(col_v, [idx])

    vec_v[...] = acc
    pltpu.sync_copy(vec_v, shared_acc.at[pl.ds(s * LANES, LANES)])
    plsc.subcore_barrier()

    @pl.when(s == 0)
    def _finish():
        pltpu.sync_copy(shared_acc, accall_v)
        tot = accall_v[pl.ds(0, LANES)]
        for t in range(1, NS):
            tot = tot + accall_v[pl.ds(t * LANES, LANES)]
        total = jnp.sum(tot) * SCALE
        vec_v[...] = jnp.full((LANES,), total, jnp.float32)
        pltpu.sync_copy(vec_v, out_hbm)


def kernel(router_probs, selected_experts):
    sel = selected_experts.astype(jnp.int32)
    colsum8 = _colsum_tc(router_probs)
    out = _lb_gather_kernel(colsum8.reshape(-1), sel[:, 0], sel[:, 1])
    return out[0] * 1.0


# speedup vs baseline: 1.5023x; 1.1490x over previous
import functools

import jax
import jax.numpy as jnp
from jax import lax
from jax.experimental import pallas as pl
from jax.experimental.pallas import tpu as pltpu
from jax.experimental.pallas import tpu_sc as plsc

N = 16384
E = 64
K = 2
NS = 16
LANES = 16
SEL_PER_TILE = (N * K) // NS
SCALE = float(E) / (float(N) * K * N)
ECH = E // LANES
BR = 4096
SUB = 8


def _colsum_tc_body(probs_ref, out_ref):
    i = pl.program_id(0)

    @pl.when(i == 0)
    def _init():
        out_ref[...] = jnp.zeros_like(out_ref)

    blk = probs_ref[...].reshape(BR // SUB, SUB, E)
    out_ref[...] += jnp.sum(blk, axis=0)


_colsum_tc = pl.pallas_call(
    _colsum_tc_body,
    grid=(N // BR,),
    in_specs=[pl.BlockSpec((BR, E), lambda i: (i, 0))],
    out_specs=pl.BlockSpec((SUB, E), lambda i: (0, 0)),
    out_shape=jax.ShapeDtypeStruct((SUB, E), jnp.float32),
)


_mesh = plsc.VectorSubcoreMesh(
    core_axis_name="c", subcore_axis_name="s", num_cores=1, num_subcores=NS
)


@functools.partial(
    pl.kernel,
    out_type=jax.ShapeDtypeStruct((LANES,), jnp.float32),
    mesh=_mesh,
    compiler_params=pltpu.CompilerParams(needs_layout_passes=False),
    scratch_types=[
        pltpu.VMEM((SUB * E,), jnp.float32),
        pltpu.VMEM((SEL_PER_TILE,), jnp.int32),
        pltpu.VMEM((E,), jnp.float32),
        pltpu.VMEM((LANES,), jnp.float32),
        pltpu.VMEM((NS * LANES,), jnp.float32),
        pltpu.VMEM_SHARED((NS * LANES,), jnp.float32),
    ],
)
def _lb_gather_kernel(colsum8_hbm, sel0_hbm, sel1_hbm, out_hbm,
                      col8_v, sel_v, col_v, vec_v, accall_v, shared_acc):
    s = lax.axis_index("s")
    half = SEL_PER_TILE // 2

    pltpu.sync_copy(colsum8_hbm, col8_v)
    pltpu.sync_copy(sel0_hbm.at[pl.ds(s * half, half)],
                    sel_v.at[pl.ds(0, half)])
    pltpu.sync_copy(sel1_hbm.at[pl.ds(s * half, half)],
                    sel_v.at[pl.ds(half, half)])

    for j in range(ECH):
        cj = col8_v[pl.ds(j * LANES, LANES)]
        for r in range(1, SUB):
            cj = cj + col8_v[pl.ds(r * E + j * LANES, LANES)]
        col_v[pl.ds(j * LANES, LANES)] = cj

    @plsc.parallel_loop(0, SEL_PER_TILE // LANES,
                        carry=jnp.zeros((LANES,), jnp.float32), unroll=8)
    def acc(i, a):
        idx = sel_v[pl.ds(i * LANES, LANES)]
        return a + plsc.load_gather(col_v, [idx])

    vec_v[...] = acc
    pltpu.sync_copy(vec_v, shared_acc.at[pl.ds(s * LANES, LANES)])
    plsc.subcore_barrier()

    @pl.when(s == 0)
    def _finish():
        pltpu.sync_copy(shared_acc, accall_v)
        tot = accall_v[pl.ds(0, LANES)]
        for t in range(1, NS):
            tot = tot + accall_v[pl.ds(t * LANES, LANES)]
        total = jnp.sum(tot) * SCALE
        vec_v[...] = jnp.full((LANES,), total, jnp.float32)
        pltpu.sync_copy(vec_v, out_hbm)


def kernel(router_probs, selected_experts):
    sel = selected_experts.astype(jnp.int32)
    colsum8 = _colsum_tc(router_probs)
    out = _lb_gather_kernel(colsum8.reshape(-1), sel[:, 0], sel[:, 1])
    return out[0] * 1.0


# speedup vs baseline: 1.5274x; 1.0167x over previous
import functools

import jax
import jax.numpy as jnp
from jax import lax
from jax.experimental import pallas as pl
from jax.experimental.pallas import tpu as pltpu
from jax.experimental.pallas import tpu_sc as plsc

N = 16384
E = 64
K = 2
NS = 16
LANES = 16
SEL_PER_TILE = (N * K) // NS
SCALE = float(E) / (float(N) * K * N)
ECH = E // LANES
BR = 8192
SUB = 8


def _colsum_tc_body(probs_ref, out_ref):
    i = pl.program_id(0)

    @pl.when(i == 0)
    def _init():
        out_ref[...] = jnp.zeros_like(out_ref)

    blk = probs_ref[...].reshape(BR // SUB, SUB, E)
    out_ref[...] += jnp.sum(blk, axis=0)


_colsum_tc = pl.pallas_call(
    _colsum_tc_body,
    grid=(N // BR,),
    in_specs=[pl.BlockSpec((BR, E), lambda i: (i, 0))],
    out_specs=pl.BlockSpec((SUB, E), lambda i: (0, 0)),
    out_shape=jax.ShapeDtypeStruct((SUB, E), jnp.float32),
)


_mesh = plsc.VectorSubcoreMesh(
    core_axis_name="c", subcore_axis_name="s", num_cores=1, num_subcores=NS
)


@functools.partial(
    pl.kernel,
    out_type=jax.ShapeDtypeStruct((LANES,), jnp.float32),
    mesh=_mesh,
    compiler_params=pltpu.CompilerParams(needs_layout_passes=False),
    scratch_types=[
        pltpu.VMEM((SUB * E,), jnp.float32),
        pltpu.VMEM((SEL_PER_TILE,), jnp.int32),
        pltpu.VMEM((E,), jnp.float32),
        pltpu.VMEM((LANES,), jnp.float32),
        pltpu.VMEM((NS * LANES,), jnp.float32),
        pltpu.VMEM_SHARED((NS * LANES,), jnp.float32),
    ],
)
def _lb_gather_kernel(colsum8_hbm, sel0_hbm, sel1_hbm, out_hbm,
                      col8_v, sel_v, col_v, vec_v, accall_v, shared_acc):
    s = lax.axis_index("s")
    half = SEL_PER_TILE // 2

    pltpu.sync_copy(colsum8_hbm, col8_v)
    pltpu.sync_copy(sel0_hbm.at[pl.ds(s * half, half)],
                    sel_v.at[pl.ds(0, half)])
    pltpu.sync_copy(sel1_hbm.at[pl.ds(s * half, half)],
                    sel_v.at[pl.ds(half, half)])

    for j in range(ECH):
        cj = col8_v[pl.ds(j * LANES, LANES)]
        for r in range(1, SUB):
            cj = cj + col8_v[pl.ds(r * E + j * LANES, LANES)]
        col_v[pl.ds(j * LANES, LANES)] = cj

    @plsc.parallel_loop(0, SEL_PER_TILE // LANES,
                        carry=jnp.zeros((LANES,), jnp.float32), unroll=8)
    def acc(i, a):
        idx = sel_v[pl.ds(i * LANES, LANES)]
        return a + plsc.load_gather(col_v, [idx])

    vec_v[...] = acc
    pltpu.sync_copy(vec_v, shared_acc.at[pl.ds(s * LANES, LANES)])
    plsc.subcore_barrier()

    @pl.when(s == 0)
    def _finish():
        pltpu.sync_copy(shared_acc, accall_v)
        tot = accall_v[pl.ds(0, LANES)]
        for t in range(1, NS):
            tot = tot + accall_v[pl.ds(t * LANES, LANES)]
        total = jnp.sum(tot) * SCALE
        vec_v[...] = jnp.full((LANES,), total, jnp.float32)
        pltpu.sync_copy(vec_v, out_hbm)


def kernel(router_probs, selected_experts):
    sel = selected_experts.astype(jnp.int32)
    colsum8 = _colsum_tc(router_probs)
    out = _lb_gather_kernel(colsum8.reshape(-1), sel[:, 0], sel[:, 1])
    return out[0] * 1.0
